# Initial kernel scaffold; baseline (speedup 1.0000x reference)
import jax
import jax.numpy as jnp
from jax.experimental import pallas as pl


def kernel(x, W_router, W_shared_gate):
    raise NotImplementedError("write your pallas kernel here")


# speedup vs baseline: 1.4951x; 1.4951x over previous
import jax
import jax.numpy as jnp
from jax.experimental import pallas as pl
from jax.experimental.pallas import tpu as pltpu

_D_MODEL = 4096
_NUM_EXPERTS = 64
_BLOCK_T = 512


def _router_kernel(x_ref, w_ref, s_ref, g_ref):
    logits_all = jnp.dot(x_ref[...], w_ref[...], preferred_element_type=jnp.float32)
    logits = logits_all[:, :_NUM_EXPERTS]
    m = jnp.max(logits, axis=-1, keepdims=True)
    e = jnp.exp(logits - m)
    s_ref[...] = e / jnp.sum(e, axis=-1, keepdims=True)
    g_ref[...] = jax.nn.sigmoid(logits_all[:, _NUM_EXPERTS:_NUM_EXPERTS + 1])


def kernel(x, W_router, W_shared_gate):
    tokens, d = x.shape
    n_exp = W_router.shape[0]
    w_all = jnp.concatenate(
        [W_router, W_shared_gate,
         jnp.zeros((128 - n_exp - 1, d), dtype=x.dtype)], axis=0).T

    grid = (tokens // _BLOCK_T,)
    s, g = pl.pallas_call(
        _router_kernel,
        grid=grid,
        in_specs=[
            pl.BlockSpec((_BLOCK_T, d), lambda i: (i, 0)),
            pl.BlockSpec((d, 128), lambda i: (0, 0)),
        ],
        out_specs=[
            pl.BlockSpec((_BLOCK_T, n_exp), lambda i: (i, 0)),
            pl.BlockSpec((_BLOCK_T, 1), lambda i: (i, 0)),
        ],
        out_shape=[
            jax.ShapeDtypeStruct((tokens, n_exp), x.dtype),
            jax.ShapeDtypeStruct((tokens, 1), x.dtype),
        ],
        compiler_params=pltpu.CompilerParams(
            dimension_semantics=("arbitrary",),
        ),
    )(x, w_all)
    return (s, g)


# speedup vs baseline: 1.6372x; 1.0950x over previous
import jax
import jax.numpy as jnp
from jax.experimental import pallas as pl
from jax.experimental.pallas import tpu as pltpu

_D_MODEL = 4096
_NUM_EXPERTS = 64
_BLOCK_T = 1024


def _router_kernel(x_ref, w_ref, s_ref, g_ref):
    logits_all = jnp.dot(x_ref[...], w_ref[...], preferred_element_type=jnp.float32)
    logits = logits_all[:, :_NUM_EXPERTS]
    m = jnp.max(logits, axis=-1, keepdims=True)
    e = jnp.exp(logits - m)
    s_ref[...] = e / jnp.sum(e, axis=-1, keepdims=True)
    g_ref[...] = jax.nn.sigmoid(logits_all[:, _NUM_EXPERTS:_NUM_EXPERTS + 1])


def kernel(x, W_router, W_shared_gate):
    tokens, d = x.shape
    n_exp = W_router.shape[0]
    w_all = jnp.concatenate(
        [W_router, W_shared_gate,
         jnp.zeros((128 - n_exp - 1, d), dtype=x.dtype)], axis=0).T

    grid = (tokens // _BLOCK_T,)
    s, g = pl.pallas_call(
        _router_kernel,
        grid=grid,
        in_specs=[
            pl.BlockSpec((_BLOCK_T, d), lambda i: (i, 0)),
            pl.BlockSpec((d, 128), lambda i: (0, 0)),
        ],
        out_specs=[
            pl.BlockSpec((_BLOCK_T, n_exp), lambda i: (i, 0)),
            pl.BlockSpec((_BLOCK_T, 1), lambda i: (i, 0)),
        ],
        out_shape=[
            jax.ShapeDtypeStruct((tokens, n_exp), x.dtype),
            jax.ShapeDtypeStruct((tokens, 1), x.dtype),
        ],
        compiler_params=pltpu.CompilerParams(
            dimension_semantics=("arbitrary",),
        ),
    )(x, w_all)
    return (s, g)


# speedup vs baseline: 1.6376x; 1.0003x over previous
import jax
import jax.numpy as jnp
from jax.experimental import pallas as pl
from jax.experimental.pallas import tpu as pltpu

_D_MODEL = 4096
_NUM_EXPERTS = 64
_BLOCK_T = 1024


def _router_kernel(x_ref, w_ref, s_ref, g_ref):
    logits_all = jnp.dot(x_ref[...], w_ref[...], preferred_element_type=jnp.float32)
    logits = logits_all[:, :_NUM_EXPERTS]
    m = jnp.max(logits, axis=-1, keepdims=True)
    e = jnp.exp(logits - m)
    s_ref[...] = e / jnp.sum(e, axis=-1, keepdims=True)
    g_ref[...] = jax.nn.sigmoid(logits_all[:, _NUM_EXPERTS:_NUM_EXPERTS + 1])


def kernel(x, W_router, W_shared_gate):
    tokens, d = x.shape
    n_exp = W_router.shape[0]
    w_all = jnp.concatenate(
        [W_router, W_shared_gate,
         jnp.zeros((128 - n_exp - 1, d), dtype=x.dtype)], axis=0).T

    grid = (tokens // _BLOCK_T,)
    s, g = pl.pallas_call(
        _router_kernel,
        grid=grid,
        in_specs=[
            pl.BlockSpec((_BLOCK_T, d), lambda i: (i, 0)),
            pl.BlockSpec((d, 128), lambda i: (0, 0)),
        ],
        out_specs=[
            pl.BlockSpec((_BLOCK_T, n_exp), lambda i: (i, 0)),
            pl.BlockSpec((_BLOCK_T, 1), lambda i: (i, 0)),
        ],
        out_shape=[
            jax.ShapeDtypeStruct((tokens, n_exp), x.dtype),
            jax.ShapeDtypeStruct((tokens, 1), x.dtype),
        ],
        compiler_params=pltpu.CompilerParams(
            dimension_semantics=("parallel",),
        ),
    )(x, w_all)
    return (s, g)
